# baseline (speedup 1.0000x reference)
import functools
import jax
import jax.numpy as jnp
from jax.experimental import pallas as pl
from jax.experimental.pallas import tpu as pltpu

N = 2048
B = 512
NB = N // B
_HIGH = jax.lax.Precision.HIGHEST
_F8 = jnp.float8_e4m3fn


def _split16(v):
    hi = v.astype(jnp.bfloat16)
    lo = (v - hi.astype(jnp.float32)).astype(jnp.bfloat16)
    return hi, lo


def _split8(v):
    chunks = []
    r = v
    for _ in range(4):
        c = r.astype(_F8)
        chunks.append(c)
        r = (r - c.astype(jnp.float32)) * 16.0
    return jnp.concatenate(chunks, axis=1)


def _dot16(h, Wcat4_ref):
    hh, hl = _split16(h)
    hcat = jnp.concatenate([hh, hh, hl, hl], axis=1)
    return jnp.dot(hcat, Wcat4_ref[...], preferred_element_type=jnp.float32)


def _gcn_kernel(hid, x_ref, W1cat_ref, b1_ref, W2_ref, b2_ref, W3_ref,
                b3_ref, Wl_ref, bl_ref, out_ref, xt_scr, h_scr, m_scr,
                deg_scr):
    i = pl.program_id(0)
    ones_col = jnp.ones((B, 1), dtype=_F8)

    xb = x_ref[...].astype(jnp.bfloat16)
    xbt8 = xb.T.astype(_F8)
    xt_scr[:, pl.ds(i * B, B)] = xbt8
    hb = jnp.dot(xb, W1cat_ref[...], preferred_element_type=jnp.float32)
    h_scr[pl.ds(i * B, B), :] = hb[:, :hid] + hb[:, hid:]
    deg_part = jnp.dot(xbt8, ones_col, preferred_element_type=jnp.float32)

    @pl.when(i == 0)
    def _():
        deg_scr[...] = deg_part + 1.0

    @pl.when(i > 0)
    def _():
        deg_scr[...] += deg_part

    @pl.when(i == NB - 1)
    def _():
        dis = jax.lax.rsqrt(deg_scr[...])
        dis2 = dis * dis

        def conv_tail(hw, b_ref):
            m_scr[...] = _split8(hw * dis)
            acc = jnp.dot(xt_scr[...], m_scr[...],
                          preferred_element_type=jnp.float32)
            agg = (acc[:, :hid] + acc[:, hid:2 * hid] * (1.0 / 16.0)
                   + acc[:, 2 * hid:3 * hid] * (1.0 / 256.0)
                   + acc[:, 3 * hid:] * (1.0 / 4096.0))
            return agg * dis + hw * dis2 + b_ref[...]

        h = jax.nn.relu(conv_tail(h_scr[...], b1_ref))
        h = jax.nn.relu(conv_tail(_dot16(h, W2_ref), b2_ref))
        h = conv_tail(_dot16(h, W3_ref), b3_ref)

        pooled = jnp.mean(h, axis=0, keepdims=True)
        out_ref[...] = jnp.dot(pooled, Wl_ref[...], precision=_HIGH,
                               preferred_element_type=jnp.float32) + bl_ref[...]


def kernel(x, W1, b1, W2, b2, W3, b3, Wl, bl):
    hid = W1.shape[1]
    W1h = W1.astype(jnp.bfloat16)
    W1l = (W1 - W1h.astype(jnp.float32)).astype(jnp.bfloat16)
    W1cat = jnp.concatenate([W1h, W1l], axis=1)

    def quad(W):
        Wh = W.astype(jnp.bfloat16)
        Wl_ = (W - Wh.astype(jnp.float32)).astype(jnp.bfloat16)
        return jnp.concatenate([Wh, Wl_, Wh, Wl_], axis=0)

    W2cat4 = quad(W2)
    W3cat4 = quad(W3)
    full = lambda shape: pl.BlockSpec(shape, lambda i: (0, 0))
    out = pl.pallas_call(
        functools.partial(_gcn_kernel, hid),
        grid=(NB,),
        in_specs=[
            pl.BlockSpec((B, N), lambda i: (i, 0)),
            full((N, 2 * hid)), full((1, hid)),
            full((4 * hid, hid)), full((1, hid)),
            full((4 * hid, hid)), full((1, hid)),
            full((hid, bl.shape[0])), full((1, bl.shape[0])),
        ],
        out_specs=full((1, bl.shape[0])),
        out_shape=jax.ShapeDtypeStruct((1, bl.shape[0]), jnp.float32),
        scratch_shapes=[pltpu.VMEM((N, N), _F8),
                        pltpu.VMEM((N, hid), jnp.float32),
                        pltpu.VMEM((N, 4 * hid), _F8),
                        pltpu.VMEM((N, 1), jnp.float32)],
        compiler_params=pltpu.CompilerParams(
            dimension_semantics=("arbitrary",)),
    )(x, W1cat, b1.reshape(1, -1), W2cat4, b2.reshape(1, -1),
      W3cat4, b3.reshape(1, -1), Wl, bl.reshape(1, -1))
    return out.reshape(-1)


# speedup vs baseline: 1.0866x; 1.0866x over previous
import functools
import jax
import jax.numpy as jnp
from jax.experimental import pallas as pl
from jax.experimental.pallas import tpu as pltpu

N = 2048
B = 512
NB = N // B
_HIGH = jax.lax.Precision.HIGHEST
_F8 = jnp.float8_e4m3fn


def _split16(v):
    hi = v.astype(jnp.bfloat16)
    lo = (v - hi.astype(jnp.float32)).astype(jnp.bfloat16)
    return hi, lo


def _split8_rows(v):
    chunks = []
    r = v
    for _ in range(4):
        c = r.astype(_F8)
        chunks.append(c)
        r = (r - c.astype(jnp.float32)) * 16.0
    return jnp.concatenate(chunks, axis=0)


def _combine4_rows(acc, hid):
    return (acc[:hid, :] + acc[hid:2 * hid, :] * (1.0 / 16.0)
            + acc[2 * hid:3 * hid, :] * (1.0 / 256.0)
            + acc[3 * hid:, :] * (1.0 / 4096.0))


def _dot16_t(Wq_ref, h):
    hh, hl = _split16(h)
    hcat = jnp.concatenate([hh, hh, hl, hl], axis=0)
    return jnp.dot(Wq_ref[...], hcat, preferred_element_type=jnp.float32)


def _gcn_kernel(hid, x_ref, W1cat8_ref, b1_ref, W2q_ref, b2_ref, W3q_ref,
                b3_ref, WlT_ref, blT_ref, out_ref, x8_scr, hT_scr, mT_scr,
                deg_scr):
    i = pl.program_id(0)
    ones_row = jnp.ones((1, B), dtype=_F8)

    xb8 = x_ref[...].astype(_F8)
    x8_scr[pl.ds(i * B, B), :] = xb8
    hb = jnp.dot(xb8, W1cat8_ref[...], preferred_element_type=jnp.float32)
    hT_scr[:, pl.ds(i * B, B)] = _combine4_rows(hb.T, hid)
    deg_part = jnp.dot(ones_row, xb8, preferred_element_type=jnp.float32)

    @pl.when(i == 0)
    def _():
        deg_scr[...] = deg_part + 1.0

    @pl.when(i > 0)
    def _():
        deg_scr[...] += deg_part

    @pl.when(i == NB - 1)
    def _():
        dis = jax.lax.rsqrt(deg_scr[...])
        dis2 = dis * dis

        def conv_tail(hwT, b_ref):
            mT_scr[...] = _split8_rows(hwT * dis)
            acc = jnp.dot(mT_scr[...], x8_scr[...],
                          preferred_element_type=jnp.float32)
            agg = _combine4_rows(acc, hid)
            return agg * dis + hwT * dis2 + b_ref[...]

        h = jax.nn.relu(conv_tail(hT_scr[...], b1_ref))
        h = jax.nn.relu(conv_tail(_dot16_t(W2q_ref, h), b2_ref))
        h = conv_tail(_dot16_t(W3q_ref, h), b3_ref)

        pooled = jnp.mean(h, axis=1, keepdims=True)
        out_ref[...] = jnp.dot(WlT_ref[...], pooled, precision=_HIGH,
                               preferred_element_type=jnp.float32) + blT_ref[...]


def kernel(x, W1, b1, W2, b2, W3, b3, Wl, bl):
    hid = W1.shape[1]
    nout = bl.shape[0]

    chunks, r = [], W1
    for _ in range(4):
        c = r.astype(_F8)
        chunks.append(c)
        r = (r - c.astype(jnp.float32)) * 16.0
    W1cat8 = jnp.concatenate(chunks, axis=1)

    def quadT(W):
        Wh = W.astype(jnp.bfloat16)
        Wl_ = (W - Wh.astype(jnp.float32)).astype(jnp.bfloat16)
        return jnp.concatenate([Wh.T, Wl_.T, Wh.T, Wl_.T], axis=1)

    full = lambda shape: pl.BlockSpec(shape, lambda i: (0, 0))
    out = pl.pallas_call(
        functools.partial(_gcn_kernel, hid),
        grid=(NB,),
        in_specs=[
            pl.BlockSpec((B, N), lambda i: (i, 0)),
            full((N, 4 * hid)), full((hid, 1)),
            full((hid, 4 * hid)), full((hid, 1)),
            full((hid, 4 * hid)), full((hid, 1)),
            full((nout, hid)), full((nout, 1)),
        ],
        out_specs=full((nout, 1)),
        out_shape=jax.ShapeDtypeStruct((nout, 1), jnp.float32),
        scratch_shapes=[pltpu.VMEM((N, N), _F8),
                        pltpu.VMEM((hid, N), jnp.float32),
                        pltpu.VMEM((4 * hid, N), _F8),
                        pltpu.VMEM((1, N), jnp.float32)],
        compiler_params=pltpu.CompilerParams(
            dimension_semantics=("arbitrary",)),
    )(x, W1cat8, b1.reshape(-1, 1), quadT(W2), b2.reshape(-1, 1),
      quadT(W3), b3.reshape(-1, 1), Wl.T, bl.reshape(-1, 1))
    return out.reshape(-1)
